# baseline (speedup 1.0000x reference)
import jax
import jax.numpy as jnp
from jax import lax
from jax.experimental import pallas as pl
from jax.experimental.pallas import tpu as pltpu
from jax.experimental.pallas import tpu_sc as plsc

EMBED = 32
BATCH = 16384
HIST = 50
HPAD = 56

_NC, _NS = 2, 16
_NW = _NC * _NS
_TCB = BATCH // 128
_TPW = _TCB // _NW
_NU = HIST * _TPW
_NBUF = 4
_LA = 2
_ETR = EMBED // 8
_HTR = HPAD // 8


def _iota16():
    return lax.iota(jnp.int32, 16)


def _emb_body(x4_hbm, table_hbm, out5_hbm, idx_v, *rest):
    icols = rest[:_NBUF]
    bufs = rest[_NBUF:2 * _NBUF]
    tbufs = rest[2 * _NBUF:3 * _NBUF]
    gsems = rest[3 * _NBUF:4 * _NBUF]
    ssems = rest[4 * _NBUF:5 * _NBUF]

    wid = lax.axis_index("s") * _NC + lax.axis_index("c")
    tcb0 = wid * _TPW
    pltpu.sync_copy(x4_hbm.at[:, pl.ds(tcb0, _TPW)], idx_v)

    def build_icol(slot, h, tc):
        tr = jnp.full((16,), h // 8, jnp.int32)
        er = jnp.full((16,), h % 8, jnp.int32)
        tcv = jnp.full((16,), tc, jnp.int32)
        for jv in range(8):
            ic = _iota16() + (16 * jv)
            vals = plsc.load_gather(idx_v, [tr, tcv, er, ic])
            icols[slot][pl.ds(16 * jv, 16)] = vals

    def g_start(slot, h, tc):
        build_icol(slot, h, tc)
        pltpu.async_copy(table_hbm.at[icols[slot]], bufs[slot], gsems[slot])

    def g_wait(slot):
        pltpu.make_async_copy(
            table_hbm.at[icols[slot]], bufs[slot], gsems[slot]).wait()

    def transpose(slot):
        for tre in range(_ETR):
            for er in range(8):
                col = jnp.full((16,), 8 * tre + er, jnp.int32)
                for jv in range(8):
                    rows = _iota16() + (16 * jv)
                    vals = plsc.load_gather(bufs[slot], [rows, col])
                    tbufs[slot][tre, er, pl.ds(16 * jv, 16)] = vals

    def s_start(slot, h, tc):
        for tre in range(_ETR):
            pltpu.async_copy(
                tbufs[slot].at[tre], out5_hbm.at[h, tre, tcb0 + tc],
                ssems[slot])

    def s_wait(slot):
        pltpu.make_async_copy(
            tbufs[slot], out5_hbm.at[0, :, 0], ssems[slot]).wait()

    for tc in range(_LA):
        g_start(tc, 0, tc)

    for tc in range(_NBUF):
        fs = (tc + _LA) % _NBUF
        g_start(fs, (tc + _LA) // _NBUF, (tc + _LA) % _NBUF)
        g_wait(tc)
        transpose(tc)
        s_start(tc, 0, tc)

    def block(h, carry):
        for tc in range(_NBUF):
            fs = (tc + _LA) % _NBUF
            hf = h + (tc + _LA) // _NBUF
            tcf = (tc + _LA) % _NBUF
            g_start(fs, hf, tcf)
            g_wait(tc)
            s_wait(tc)
            transpose(tc)
            s_start(tc, h, tc)
        return carry

    lax.fori_loop(1, HIST - 1, block, 0)

    h_last = HIST - 1
    for tc in range(_NBUF):
        if tc < _NBUF - _LA:
            fs = (tc + _LA) % _NBUF
            g_start(fs, h_last + (tc + _LA) // _NBUF, (tc + _LA) % _NBUF)
        g_wait(tc)
        s_wait(tc)
        transpose(tc)
        s_start(tc, h_last, tc)

    for tc in range(_NBUF):
        s_wait(tc)


def kernel(x, table):
    xpad = jnp.pad(x, ((0, 0), (0, HPAD - HIST)))
    x4 = (xpad.T.reshape(_HTR, 8, _TCB, 128).transpose(0, 2, 1, 3))
    scratch = [pltpu.VMEM((_HTR, _TPW, 8, 128), jnp.int32)]
    scratch += [pltpu.VMEM((128,), jnp.int32) for _ in range(_NBUF)]
    scratch += [pltpu.VMEM((128, EMBED), jnp.float32) for _ in range(_NBUF)]
    scratch += [pltpu.VMEM((_ETR, 8, 128), jnp.float32) for _ in range(_NBUF)]
    scratch += [pltpu.SemaphoreType.DMA for _ in range(2 * _NBUF)]
    out5 = pl.kernel(
        _emb_body,
        out_type=jax.ShapeDtypeStruct((HIST, _ETR, _TCB, 8, 128),
                                      jnp.float32),
        scratch_types=scratch,
        mesh=plsc.VectorSubcoreMesh(core_axis_name="c", subcore_axis_name="s"),
        compiler_params=pltpu.CompilerParams(use_tc_tiling_on_sc=False,
                                             needs_layout_passes=False),
    )(x4, table)
    return out5.transpose(2, 4, 0, 1, 3).reshape(BATCH, HIST, EMBED)


# speedup vs baseline: 1.2554x; 1.2554x over previous
import jax
import jax.numpy as jnp
from jax import lax
from jax.experimental import pallas as pl
from jax.experimental.pallas import tpu as pltpu
from jax.experimental.pallas import tpu_sc as plsc

EMBED = 32
BATCH = 16384
HIST = 50
HPAD = 56

_NC, _NS = 2, 16
_NW = _NC * _NS
_TCB = BATCH // 128
_TPW = _TCB // _NW
_NU = HIST * _TPW
_NBUF = 4
_LA = 2
_ETR = EMBED // 8
_HTR = HPAD // 8


def _iota16():
    return lax.iota(jnp.int32, 16)


def _emb_body(x4_hbm, table_hbm, out5_hbm, idx_v, *rest):
    icols = rest[:_NBUF]
    bufs = rest[_NBUF:2 * _NBUF]
    tbufs = rest[2 * _NBUF:3 * _NBUF]
    gsems = rest[3 * _NBUF:4 * _NBUF]
    ssems = rest[4 * _NBUF:5 * _NBUF]

    wid = lax.axis_index("s") * _NC + lax.axis_index("c")
    tcb0 = wid * _TPW
    pltpu.sync_copy(x4_hbm.at[:, pl.ds(tcb0, _TPW)], idx_v)
    ec0 = lax.iota(jnp.int32, 16)
    ec1 = ec0 + 16

    def build_icol(slot, h, tc):
        tr = jnp.full((16,), h // 8, jnp.int32)
        er = jnp.full((16,), h % 8, jnp.int32)
        tcv = jnp.full((16,), tc, jnp.int32)
        for jv in range(8):
            ic = _iota16() + (16 * jv)
            vals = plsc.load_gather(idx_v, [tr, tcv, er, ic])
            icols[slot][pl.ds(16 * jv, 16)] = vals

    def g_start(slot, h, tc):
        build_icol(slot, h, tc)
        pltpu.async_copy(table_hbm.at[icols[slot]], bufs[slot], gsems[slot])

    def g_wait(slot):
        pltpu.make_async_copy(
            table_hbm.at[icols[slot]], bufs[slot], gsems[slot]).wait()

    def transpose(slot):
        def tr_body(i, carry):
            r0 = i * 8
            for k in range(8):
                r = r0 + k
                rv = jnp.full((16,), r, jnp.int32)
                v0 = bufs[slot][r, pl.ds(0, 16)]
                v1 = bufs[slot][r, pl.ds(16, 16)]
                plsc.store_scatter(tbufs[slot], [ec0, rv], v0)
                plsc.store_scatter(tbufs[slot], [ec1, rv], v1)
            return carry

        lax.fori_loop(0, 16, tr_body, 0)

    def s_start(slot, h, tc):
        for tre in range(_ETR):
            pltpu.async_copy(
                tbufs[slot].at[pl.ds(8 * tre, 8)],
                out5_hbm.at[h, tre, tcb0 + tc], ssems[slot])

    def s_wait(slot):
        for tre in range(_ETR):
            pltpu.make_async_copy(
                tbufs[slot].at[pl.ds(8 * tre, 8)],
                out5_hbm.at[0, tre, 0], ssems[slot]).wait()

    for tc in range(_LA):
        g_start(tc, 0, tc)

    for tc in range(_NBUF):
        fs = (tc + _LA) % _NBUF
        g_start(fs, (tc + _LA) // _NBUF, (tc + _LA) % _NBUF)
        g_wait(tc)
        transpose(tc)
        s_start(tc, 0, tc)

    def block(h, carry):
        for tc in range(_NBUF):
            fs = (tc + _LA) % _NBUF
            hf = h + (tc + _LA) // _NBUF
            tcf = (tc + _LA) % _NBUF
            g_start(fs, hf, tcf)
            g_wait(tc)
            s_wait(tc)
            transpose(tc)
            s_start(tc, h, tc)
        return carry

    lax.fori_loop(1, HIST - 1, block, 0)

    h_last = HIST - 1
    for tc in range(_NBUF):
        if tc < _NBUF - _LA:
            fs = (tc + _LA) % _NBUF
            g_start(fs, h_last + (tc + _LA) // _NBUF, (tc + _LA) % _NBUF)
        g_wait(tc)
        s_wait(tc)
        transpose(tc)
        s_start(tc, h_last, tc)

    for tc in range(_NBUF):
        s_wait(tc)


def kernel(x, table):
    xpad = jnp.pad(x, ((0, 0), (0, HPAD - HIST)))
    x4 = (xpad.T.reshape(_HTR, 8, _TCB, 128).transpose(0, 2, 1, 3))
    scratch = [pltpu.VMEM((_HTR, _TPW, 8, 128), jnp.int32)]
    scratch += [pltpu.VMEM((128,), jnp.int32) for _ in range(_NBUF)]
    scratch += [pltpu.VMEM((128, EMBED), jnp.float32) for _ in range(_NBUF)]
    scratch += [pltpu.VMEM((EMBED, 128), jnp.float32) for _ in range(_NBUF)]
    scratch += [pltpu.SemaphoreType.DMA for _ in range(2 * _NBUF)]
    out5 = pl.kernel(
        _emb_body,
        out_type=jax.ShapeDtypeStruct((HIST, _ETR, _TCB, 8, 128),
                                      jnp.float32),
        scratch_types=scratch,
        mesh=plsc.VectorSubcoreMesh(core_axis_name="c", subcore_axis_name="s"),
        compiler_params=pltpu.CompilerParams(use_tc_tiling_on_sc=False,
                                             needs_layout_passes=False),
    )(x4, table)
    return out5.transpose(2, 4, 0, 1, 3).reshape(BATCH, HIST, EMBED)


# speedup vs baseline: 1.3629x; 1.0856x over previous
import jax
import jax.numpy as jnp
from jax import lax
from jax.experimental import pallas as pl
from jax.experimental.pallas import tpu as pltpu
from jax.experimental.pallas import tpu_sc as plsc

EMBED = 32
BATCH = 16384
HIST = 50
HPAD = 56

_NC, _NS = 2, 16
_NW = _NC * _NS
_TCB = BATCH // 128
_TPW = _TCB // _NW
_NU = HIST * _TPW
_NBUF = 4
_LA = 2
_ETR = EMBED // 8
_HTR = HPAD // 8


def _iota16():
    return lax.iota(jnp.int32, 16)


def _emb_body(x4_hbm, table_hbm, out5_hbm, idx_v, *rest):
    icols = rest[:_NBUF]
    bufs = rest[_NBUF:2 * _NBUF]
    tbufs = rest[2 * _NBUF:3 * _NBUF]
    gsems = rest[3 * _NBUF:4 * _NBUF]
    ssems = rest[4 * _NBUF:5 * _NBUF]

    wid = lax.axis_index("s") * _NC + lax.axis_index("c")
    tcb0 = wid * _TPW
    pltpu.sync_copy(x4_hbm.at[:, pl.ds(tcb0, _TPW)], idx_v)
    ec0 = lax.iota(jnp.int32, 16)
    ec1 = ec0 + 16

    def build_icol(slot, h, tc):
        tr = jnp.full((16,), h // 8, jnp.int32)
        er = jnp.full((16,), h % 8, jnp.int32)
        tcv = jnp.full((16,), tc, jnp.int32)
        for jv in range(8):
            ic = _iota16() + (16 * jv)
            vals = plsc.load_gather(idx_v, [tr, tcv, er, ic])
            icols[slot][pl.ds(16 * jv, 16)] = vals

    def g_start(slot, h, tc):
        build_icol(slot, h, tc)
        pltpu.async_copy(table_hbm.at[icols[slot]], bufs[slot], gsems[slot])

    def g_wait(slot):
        pltpu.make_async_copy(
            table_hbm.at[icols[slot]], bufs[slot], gsems[slot]).wait()

    def transpose(slot):
        @plsc.parallel_loop(0, 128, step=8, unroll=2)
        def tr_body(i):
            for k in range(8):
                r = i + k
                rv = jnp.full((16,), r, jnp.int32)
                v0 = bufs[slot][r, pl.ds(0, 16)]
                v1 = bufs[slot][r, pl.ds(16, 16)]
                plsc.store_scatter(tbufs[slot], [ec0, rv], v0)
                plsc.store_scatter(tbufs[slot], [ec1, rv], v1)

    def s_start(slot, h, tc):
        for tre in range(_ETR):
            pltpu.async_copy(
                tbufs[slot].at[pl.ds(8 * tre, 8)],
                out5_hbm.at[h, tre, tcb0 + tc], ssems[slot])

    def s_wait(slot):
        for tre in range(_ETR):
            pltpu.make_async_copy(
                tbufs[slot].at[pl.ds(8 * tre, 8)],
                out5_hbm.at[0, tre, 0], ssems[slot]).wait()

    for tc in range(_LA):
        g_start(tc, 0, tc)

    for tc in range(_NBUF):
        fs = (tc + _LA) % _NBUF
        g_start(fs, (tc + _LA) // _NBUF, (tc + _LA) % _NBUF)
        g_wait(tc)
        transpose(tc)
        s_start(tc, 0, tc)

    def block(h, carry):
        for tc in range(_NBUF):
            fs = (tc + _LA) % _NBUF
            hf = h + (tc + _LA) // _NBUF
            tcf = (tc + _LA) % _NBUF
            g_start(fs, hf, tcf)
            g_wait(tc)
            s_wait(tc)
            transpose(tc)
            s_start(tc, h, tc)
        return carry

    lax.fori_loop(1, HIST - 1, block, 0)

    h_last = HIST - 1
    for tc in range(_NBUF):
        if tc < _NBUF - _LA:
            fs = (tc + _LA) % _NBUF
            g_start(fs, h_last + (tc + _LA) // _NBUF, (tc + _LA) % _NBUF)
        g_wait(tc)
        s_wait(tc)
        transpose(tc)
        s_start(tc, h_last, tc)

    for tc in range(_NBUF):
        s_wait(tc)


def kernel(x, table):
    xpad = jnp.pad(x, ((0, 0), (0, HPAD - HIST)))
    x4 = (xpad.T.reshape(_HTR, 8, _TCB, 128).transpose(0, 2, 1, 3))
    scratch = [pltpu.VMEM((_HTR, _TPW, 8, 128), jnp.int32)]
    scratch += [pltpu.VMEM((128,), jnp.int32) for _ in range(_NBUF)]
    scratch += [pltpu.VMEM((128, EMBED), jnp.float32) for _ in range(_NBUF)]
    scratch += [pltpu.VMEM((EMBED, 128), jnp.float32) for _ in range(_NBUF)]
    scratch += [pltpu.SemaphoreType.DMA for _ in range(2 * _NBUF)]
    out5 = pl.kernel(
        _emb_body,
        out_type=jax.ShapeDtypeStruct((HIST, _ETR, _TCB, 8, 128),
                                      jnp.float32),
        scratch_types=scratch,
        mesh=plsc.VectorSubcoreMesh(core_axis_name="c", subcore_axis_name="s"),
        compiler_params=pltpu.CompilerParams(use_tc_tiling_on_sc=False,
                                             needs_layout_passes=False),
    )(x4, table)
    return out5.transpose(2, 4, 0, 1, 3).reshape(BATCH, HIST, EMBED)
